# baseline (speedup 1.0000x reference)
import functools

import jax
import jax.numpy as jnp
from jax import lax
from jax.experimental import pallas as pl
from jax.experimental.pallas import tpu as pltpu
from jax.experimental.pallas import tpu_sc as plsc

_N = 32
_F = 3 * 512 * 512
_TOTAL = _N * _F
_COPY = (_N - 1) * _F
_NC = 2
_CHUNK = _COPY // _NC
_XCHUNK = _F // _NC
_NBUF = 4
_K = 31
_B = _CHUNK // _K

_mesh = plsc.ScalarSubcoreMesh(axis_name="c")


@functools.partial(
    pl.kernel,
    mesh=_mesh,
    out_type=jax.ShapeDtypeStruct((_TOTAL,), jnp.float32),
    scratch_types=(
        [pltpu.VMEM_SHARED((_B,), jnp.float32) for _ in range(_NBUF)]
        + [pltpu.VMEM_SHARED((_XCHUNK,), jnp.float32)]
        + [pltpu.SemaphoreType.DMA for _ in range(2 * _NBUF + 1)]
    ),
)
def _ring_update(x_hbm, t_hbm, out_hbm, *scratch):
    bufs = scratch[:_NBUF]
    xbuf = scratch[_NBUF]
    isems = scratch[_NBUF + 1:2 * _NBUF + 1]
    osems = scratch[2 * _NBUF + 1:3 * _NBUF + 1]
    sx = scratch[3 * _NBUF + 1]

    wid = lax.axis_index("c")
    base = pl.multiple_of(wid * _CHUNK, 8)
    xb = pl.multiple_of(wid * _XCHUNK, 8)

    def in_copy(k):
        s = k % _NBUF
        return pltpu.make_async_copy(
            t_hbm.at[pl.ds(_F + base + k * _B, _B)], bufs[s], isems[s])

    def out_copy(k):
        s = k % _NBUF
        return pltpu.make_async_copy(
            bufs[s], out_hbm.at[pl.ds(base + k * _B, _B)], osems[s])

    x_in = pltpu.make_async_copy(x_hbm.at[pl.ds(xb, _XCHUNK)], xbuf, sx)
    x_in.start()

    for j in range(_NBUF - 1):
        in_copy(j).start()
    for k in range(_K):
        if k + _NBUF - 1 < _K:
            if k >= 1:
                out_copy(k - 1).wait()
            in_copy(k + _NBUF - 1).start()
        in_copy(k).wait()
        out_copy(k).start()

    x_in.wait()
    x_out = pltpu.make_async_copy(
        xbuf, out_hbm.at[pl.ds(_COPY + xb, _XCHUNK)], sx)
    x_out.start()
    for k in range(max(0, _K - _NBUF), _K):
        out_copy(k).wait()
    x_out.wait()


def kernel(x, tensors):
    out = _ring_update(x.reshape(-1), tensors.reshape(-1))
    return out.reshape(tensors.shape)


# speedup vs baseline: 4.1909x; 4.1909x over previous
import jax
import jax.numpy as jnp
from jax.experimental import pallas as pl
from jax.experimental.pallas import tpu as pltpu

_N = 32
_R = 3 * 512
_W = 512


def _body(x_ref, t_ref, o_ref):
    i = pl.program_id(0)

    @pl.when(i < _N - 1)
    def _():
        o_ref[...] = t_ref[...]

    @pl.when(i == _N - 1)
    def _():
        o_ref[...] = x_ref[...]


def kernel(x, tensors):
    t2 = tensors.reshape(_N * _R, _W)
    x2 = x.reshape(_R, _W)
    out = pl.pallas_call(
        _body,
        grid=(_N,),
        in_specs=[
            pl.BlockSpec((_R, _W), lambda i: (0, 0)),
            pl.BlockSpec((_R, _W), lambda i: (jnp.minimum(i + 1, _N - 1), 0)),
        ],
        out_specs=pl.BlockSpec((_R, _W), lambda i: (i, 0)),
        out_shape=jax.ShapeDtypeStruct((_N * _R, _W), jnp.float32),
    )(x2, t2)
    return out.reshape(tensors.shape)
